# Initial kernel scaffold; baseline (speedup 1.0000x reference)
import jax
import jax.numpy as jnp
from jax.experimental import pallas as pl


def kernel(x, edge_index, W1, b1, W2, b2):
    raise NotImplementedError("write your pallas kernel here")


# speedup vs baseline: 9.0624x; 9.0624x over previous
import jax
import jax.numpy as jnp
from jax import lax
from jax.experimental import pallas as pl
from jax.experimental.pallas import tpu as pltpu
from jax.experimental.pallas import tpu_sc as plsc

N = 10000
D = 128
NC = 2
NS = 16
NW = NC * NS
CH = 128
NP = 10112
RPT = NP // NS
NBUF = 2


def _mesh():
  return plsc.VectorSubcoreMesh(
      core_axis_name="c", subcore_axis_name="s", num_cores=NC, num_subcores=NS)


def _zero_rows(ref, nrows):
  cols = ref.shape[1] // 16
  @pl.loop(0, nrows)
  def _(r):
    for l in range(cols):
      ref[r, pl.ds(l * 16, 16)] = jnp.zeros((16,), jnp.float32)


def _unpack(pk, n, srcb, dstb):
  for l in range(CH // 16):
    p = pk[n, pl.ds(l * 16, 16)]
    if srcb is not None:
      srcb[pl.ds(l * 16, 16)] = lax.bitwise_and(p, 0x3FFF)
    dstb[pl.ds(l * 16, 16)] = lax.shift_right_logical(p, 14)


CHD = 64


def _unpack_dst64(pk, row, half, dstb):
  for l in range(CHD // 16):
    p = pk[row, pl.ds(half * CHD + l * 16, 16)]
    dstb[pl.ds(l * 16, 16)] = lax.shift_right_logical(p, 14)


def _deg_kernel(nch):
  assert RPT == 9 * CHD + 56
  def body(pk3, out, pkv, dstb, ones, zsrc, acc, sems):
    c = lax.axis_index("c")
    s = lax.axis_index("s")
    wid = c * NS + s
    @pl.loop(0, CHD)
    def _(r):
      for l in range(D // 16):
        ones[r, pl.ds(l * 16, 16)] = jnp.ones((16,), jnp.float32)
    _zero_rows(zsrc, CHD)
    pltpu.sync_copy(pk3.at[wid], pkv)
    for k in range(9):
      pltpu.sync_copy(zsrc, acc.at[pl.ds(s * RPT + k * CHD, CHD)])
    pltpu.sync_copy(zsrc.at[pl.ds(0, 56)],
                    acc.at[pl.ds(s * RPT + 9 * CHD, 56)])
    plsc.subcore_barrier()
    nchd = nch * (CH // CHD)
    @pl.loop(0, nchd // 2)
    def _(cc):
      for b in range(2):
        m = cc * 2 + b
        @pl.when(m >= 2)
        def _():
          pltpu.make_async_copy(ones, acc.at[dstb[b]], sems[b]).wait()
        _unpack_dst64(pkv, cc, b, dstb[b])
        pltpu.async_copy(ones, acc.at[dstb[b]], sems[b], add=True)
    for b in range(2):
      pltpu.make_async_copy(ones, acc.at[dstb[b]], sems[b]).wait()
    plsc.subcore_barrier()
    for k in range(9):
      pltpu.sync_copy(acc.at[pl.ds(s * RPT + k * CHD, CHD)], zsrc)
      pltpu.sync_copy(zsrc, out.at[c, pl.ds(s * RPT + k * CHD, CHD)])
    pltpu.sync_copy(acc.at[pl.ds(s * RPT + 9 * CHD, 56)],
                    zsrc.at[pl.ds(0, 56)])
    pltpu.sync_copy(zsrc.at[pl.ds(0, 56)],
                    out.at[c, pl.ds(s * RPT + 9 * CHD, 56)])

  return pl.kernel(
      body,
      out_type=jax.ShapeDtypeStruct((NC, NP, D), jnp.float32),
      mesh=_mesh(),
      scratch_types=[
          pltpu.VMEM((nch, CH), jnp.int32),
          [pltpu.VMEM((CHD,), jnp.int32) for _ in range(2)],
          pltpu.VMEM((CHD, D), jnp.float32),
          pltpu.VMEM((CHD, D), jnp.float32),
          pltpu.VMEM_SHARED((NP, D), jnp.float32),
          [pltpu.SemaphoreType.DMA for _ in range(2)],
      ],
  )


def _agg_kernel(nch):
  assert RPT == 4 * CH + 120 and nch % NBUF == 0 and NBUF == 2
  def body(pk3, g, out, pkv, srcb, dstb, rows, acc, gsems, ssems):
    c = lax.axis_index("c")
    s = lax.axis_index("s")
    wid = c * NS + s
    _zero_rows(rows[0], CH)
    for k in range(4):
      pltpu.sync_copy(rows[0], acc.at[pl.ds(s * RPT + k * CH, CH)])
    pltpu.sync_copy(rows[0].at[pl.ds(0, 120)],
                    acc.at[pl.ds(s * RPT + 4 * CH, 120)])
    pltpu.sync_copy(pk3.at[wid], pkv)
    plsc.subcore_barrier()
    for b in range(NBUF):
      _unpack(pkv, b, srcb[b], dstb[b])
      pltpu.async_copy(g.at[srcb[b]], rows[b], gsems[b])
    @pl.loop(0, nch // NBUF)
    def _(cc):
      for b in range(NBUF):
        n = cc * NBUF + b
        pb = (b - 1) % NBUF
        @pl.when(n >= 1)
        def _():
          pltpu.make_async_copy(
              rows[pb], acc.at[dstb[pb]], ssems[pb]).wait()
          @pl.when(n + 1 < nch)
          def _():
            _unpack(pkv, n + 1, srcb[pb], dstb[pb])
            pltpu.async_copy(g.at[srcb[pb]], rows[pb], gsems[pb])
        pltpu.make_async_copy(g.at[srcb[b]], rows[b], gsems[b]).wait()
        pltpu.async_copy(rows[b], acc.at[dstb[b]], ssems[b], add=True)
    lastb = (nch - 1) % NBUF
    pltpu.make_async_copy(
        rows[lastb], acc.at[dstb[lastb]], ssems[lastb]).wait()
    plsc.subcore_barrier()
    for k in range(4):
      pltpu.sync_copy(acc.at[pl.ds(s * RPT + k * CH, CH)], rows[k % 2])
      pltpu.sync_copy(rows[k % 2], out.at[c, pl.ds(s * RPT + k * CH, CH)])
    pltpu.sync_copy(acc.at[pl.ds(s * RPT + 4 * CH, 120)],
                    rows[0].at[pl.ds(0, 120)])
    pltpu.sync_copy(rows[0].at[pl.ds(0, 120)],
                    out.at[c, pl.ds(s * RPT + 4 * CH, 120)])

  return pl.kernel(
      body,
      out_type=jax.ShapeDtypeStruct((NC, NP, D), jnp.float32),
      mesh=_mesh(),
      scratch_types=[
          pltpu.VMEM((nch, CH), jnp.int32),
          [pltpu.VMEM((CH,), jnp.int32) for _ in range(NBUF)],
          [pltpu.VMEM((CH,), jnp.int32) for _ in range(NBUF)],
          [pltpu.VMEM((CH, D), jnp.float32) for _ in range(NBUF)],
          pltpu.VMEM_SHARED((NP, D), jnp.float32),
          [pltpu.SemaphoreType.DMA for _ in range(NBUF)],
          [pltpu.SemaphoreType.DMA for _ in range(NBUF)],
      ],
  )


def _dis_body(degp_ref, dis_ref):
  deg = degp_ref[0, :, 0:1] + degp_ref[1, :, 0:1] + 1.0
  dis_ref[...] = lax.rsqrt(deg)


def _mm_scale_body(x_ref, w_ref, dis_ref, o_ref):
  h = jnp.dot(x_ref[...], w_ref[...], preferred_element_type=jnp.float32)
  o_ref[...] = h * dis_ref[...]


def _layer_body(acc_ref, g_ref, dis_ref, b_ref, w_ref, o_ref):
  conv = (acc_ref[0] + acc_ref[1] + g_ref[...]) * dis_ref[...] + b_ref[...]
  h = jnp.maximum(conv, 0.0)
  o_ref[...] = jnp.dot(h, w_ref[...],
                       preferred_element_type=jnp.float32) * dis_ref[...]


def _out_body(acc_ref, g_ref, dis_ref, b_ref, o_ref):
  y = (acc_ref[0] + acc_ref[1] + g_ref[...]) * dis_ref[...] + b_ref[...]
  m = jnp.max(y, axis=1, keepdims=True)
  lse = m + jnp.log(jnp.sum(jnp.exp(y - m), axis=1, keepdims=True))
  o_ref[...] = y - lse


def _tc_call(body, n_rows, br, out_d, in_specs):
  return pl.pallas_call(
      body,
      grid=(n_rows // br,),
      in_specs=in_specs,
      out_specs=pl.BlockSpec((br, out_d), lambda i: (i, 0)),
      out_shape=jax.ShapeDtypeStruct((n_rows, out_d), jnp.float32),
  )


def kernel(x, edge_index, W1, b1, W2, b2):
  n, d = x.shape
  assert (n, d) == (N, D)
  e = edge_index.shape[1]
  ei = edge_index.astype(jnp.int32)
  ept = -(-e // (NW * CH * NBUF)) * CH * NBUF
  nch = ept // CH
  pad = NW * ept - e
  src_p = jnp.concatenate([ei[0], jnp.zeros((pad,), jnp.int32)])
  dst_p = jnp.concatenate([ei[1], jnp.full((pad,), N, jnp.int32)])
  pk3 = (src_p | (dst_p << 14)).reshape(NW, nch, CH)

  degp = _deg_kernel(nch)(pk3)
  dis_np = pl.pallas_call(
      _dis_body,
      out_shape=jax.ShapeDtypeStruct((NP, 1), jnp.float32),
  )(degp)
  dis = dis_np[:N]

  br = 1000
  b1r = b1.reshape(1, D)
  b2r = b2.reshape(1, D)
  w_spec = pl.BlockSpec((D, D), lambda i: (0, 0))
  dis_spec = pl.BlockSpec((br, 1), lambda i: (i, 0))
  row_spec = pl.BlockSpec((br, D), lambda i: (i, 0))
  acc_spec = pl.BlockSpec((NC, br, D), lambda i: (0, i, 0))
  b_spec = pl.BlockSpec((1, D), lambda i: (0, 0))

  g1 = _tc_call(_mm_scale_body, N, br, D,
                [row_spec, w_spec, dis_spec])(x, W1, dis)
  acc1 = _agg_kernel(nch)(pk3, g1)[:, :N, :]
  g2 = _tc_call(_layer_body, N, br, D,
                [acc_spec, row_spec, dis_spec, b_spec, w_spec])(
                    acc1, g1, dis, b1r, W2)
  acc2 = _agg_kernel(nch)(pk3, g2)[:, :N, :]
  out = _tc_call(_out_body, N, br, D,
                 [acc_spec, row_spec, dis_spec, b_spec])(acc2, g2, dis, b2r)
  return out
